# baseline (speedup 1.0000x reference)
import functools

import jax
import jax.numpy as jnp
from jax import lax
from jax.experimental import pallas as pl
from jax.experimental.pallas import tpu as pltpu
from jax.experimental.pallas import tpu_sc as plsc

_LANES = 16
_U_CHUNKS = 4


def _booleanize_sc(mask_flat):
    n = mask_flat.shape[0]
    info = plsc.get_sparse_core_info()
    nc, ns = info.num_cores, info.num_subcores
    nw = nc * ns
    per_w = n // nw
    assert per_w % _LANES == 0 and n % nw == 0

    mesh = plsc.VectorSubcoreMesh(core_axis_name="c", subcore_axis_name="s")

    @functools.partial(
        pl.kernel,
        mesh=mesh,
        out_type=jax.ShapeDtypeStruct((n,), jnp.int32),
        compiler_params=pltpu.CompilerParams(needs_layout_passes=False),
        scratch_types=[
            pltpu.VMEM((per_w,), jnp.float32),
            pltpu.VMEM((per_w,), jnp.int32),
        ],
    )
    def k(m_hbm, out_hbm, m_v, o_v):
        wid = lax.axis_index("s") * nc + lax.axis_index("c")
        base = wid * per_w
        pltpu.sync_copy(m_hbm.at[pl.ds(base, per_w)], m_v)
        for i in range(per_w // _LANES):
            sl = pl.ds(i * _LANES, _LANES)
            o_v[sl] = (m_v[sl] != 0.0).astype(jnp.int32)
        pltpu.sync_copy(o_v, out_hbm.at[pl.ds(base, per_w)])

    return k(mask_flat)


_NBUF = 8
_LAG = 4


def _copy_u_tc(U):
    n = U.size
    Uf = U.reshape(n)
    ch = n // 64
    n_chunks = n // ch

    def body(u_in, u_out, buf, in_sems, out_sems):
        def in_cp(c):
            k = c % _NBUF
            return pltpu.make_async_copy(
                u_in.at[pl.ds(c * ch, ch)], buf.at[k], in_sems.at[k]
            )

        def out_cp(c):
            k = c % _NBUF
            return pltpu.make_async_copy(
                buf.at[k], u_out.at[pl.ds(c * ch, ch)], out_sems.at[k]
            )

        for c in range(n_chunks + _LAG):
            if c < n_chunks:
                if c >= _NBUF:
                    out_cp(c - _NBUF).wait()
                in_cp(c).start()
            d = c - _LAG
            if d >= 0:
                in_cp(d).wait()
                out_cp(d).start()
        for c in range(n_chunks - _NBUF, n_chunks):
            out_cp(c).wait()

    out = pl.pallas_call(
        body,
        in_specs=[pl.BlockSpec(memory_space=pl.ANY)],
        out_specs=pl.BlockSpec(memory_space=pl.ANY),
        out_shape=jax.ShapeDtypeStruct(Uf.shape, Uf.dtype),
        scratch_shapes=[
            pltpu.VMEM((_NBUF, ch), jnp.float32),
            pltpu.SemaphoreType.DMA((_NBUF,)),
            pltpu.SemaphoreType.DMA((_NBUF,)),
        ],
        compiler_params=pltpu.CompilerParams(
            vmem_limit_bytes=100 * 1024 * 1024,
        ),
    )(Uf)
    return out.reshape(U.shape)


def kernel(x, v, mask, U):
    mi = _booleanize_sc(mask.reshape(-1))
    oU = _copy_u_tc(U)
    mb = mi.astype(jnp.bool_).reshape(mask.shape)
    return (x, v, mb, oU)


# speedup vs baseline: 2.9354x; 2.9354x over previous
import functools

import jax
import jax.numpy as jnp
from jax import lax
from jax.experimental import pallas as pl
from jax.experimental.pallas import tpu as pltpu
from jax.experimental.pallas import tpu_sc as plsc

_LANES = 16
_X_CHUNK = 32768


def _sc_part(mask_flat, x_flat, v_flat):
    n = mask_flat.shape[0]
    nx = x_flat.shape[0]
    nv = v_flat.shape[0]
    info = plsc.get_sparse_core_info()
    nc, ns = info.num_cores, info.num_subcores
    nw = nc * ns
    per_w = n // nw
    x_w = nx // nw
    v_w = nv // nw
    assert per_w % _LANES == 0 and n % nw == 0
    assert nx % (nw * _X_CHUNK) == 0 and nv % nw == 0

    mesh = plsc.VectorSubcoreMesh(core_axis_name="c", subcore_axis_name="s")

    @functools.partial(
        pl.kernel,
        mesh=mesh,
        out_type=[
            jax.ShapeDtypeStruct((n,), jnp.int32),
            jax.ShapeDtypeStruct((nx,), jnp.float32),
            jax.ShapeDtypeStruct((nv,), jnp.float32),
        ],
        compiler_params=pltpu.CompilerParams(needs_layout_passes=False),
        scratch_types=[
            pltpu.VMEM((per_w,), jnp.float32),
            pltpu.VMEM((per_w,), jnp.int32),
            pltpu.VMEM((_X_CHUNK,), jnp.float32),
            pltpu.VMEM((v_w,), jnp.float32),
        ],
    )
    def k(m_hbm, x_hbm, v_hbm, mi_hbm, ox_hbm, ov_hbm, m_v, o_v, x_v, v_v):
        wid = lax.axis_index("s") * nc + lax.axis_index("c")
        base = wid * per_w
        pltpu.sync_copy(m_hbm.at[pl.ds(base, per_w)], m_v)
        for i in range(per_w // _LANES):
            sl = pl.ds(i * _LANES, _LANES)
            o_v[sl] = (m_v[sl] != 0.0).astype(jnp.int32)
        pltpu.sync_copy(o_v, mi_hbm.at[pl.ds(base, per_w)])

        vbase = wid * v_w
        pltpu.sync_copy(v_hbm.at[pl.ds(vbase, v_w)], v_v)
        pltpu.sync_copy(v_v, ov_hbm.at[pl.ds(vbase, v_w)])

        xbase = wid * x_w
        for p in range(x_w // _X_CHUNK):
            sl = pl.ds(xbase + p * _X_CHUNK, _X_CHUNK)
            pltpu.sync_copy(x_hbm.at[sl], x_v)
            pltpu.sync_copy(x_v, ox_hbm.at[sl])

    return k(mask_flat, x_flat, v_flat)


def _copy_u_tc(U):
    R = 8
    Uf = U.reshape(-1, U.shape[-2], U.shape[-1])
    n = Uf.shape[0] // R

    def body(u_in, u_out):
        u_out[...] = u_in[...]

    out = pl.pallas_call(
        body,
        grid=(n,),
        in_specs=[pl.BlockSpec((R, 512, 512), lambda i: (i, 0, 0))],
        out_specs=pl.BlockSpec((R, 512, 512), lambda i: (i, 0, 0)),
        out_shape=jax.ShapeDtypeStruct(Uf.shape, Uf.dtype),
        compiler_params=pltpu.CompilerParams(
            dimension_semantics=("parallel",),
            vmem_limit_bytes=100 * 1024 * 1024,
        ),
    )(Uf)
    return out.reshape(U.shape)


def kernel(x, v, mask, U):
    mi, ox, ov = _sc_part(mask.reshape(-1), x.reshape(-1), v.reshape(-1))
    oU = _copy_u_tc(U)
    mb = mi.astype(jnp.bool_).reshape(mask.shape)
    return (ox.reshape(x.shape), ov.reshape(v.shape), mb, oU)


# speedup vs baseline: 3.1526x; 1.0740x over previous
import functools

import jax
import jax.numpy as jnp
from jax import lax
from jax.experimental import pallas as pl
from jax.experimental.pallas import tpu as pltpu
from jax.experimental.pallas import tpu_sc as plsc

_LANES = 16
_NBUF = 8
_LAG = 4


def _booleanize_sc(mask_flat):
    n = mask_flat.shape[0]
    info = plsc.get_sparse_core_info()
    nc, ns = info.num_cores, info.num_subcores
    nw = nc * ns
    per_w = n // nw
    assert per_w % _LANES == 0 and n % nw == 0

    mesh = plsc.VectorSubcoreMesh(core_axis_name="c", subcore_axis_name="s")

    @functools.partial(
        pl.kernel,
        mesh=mesh,
        out_type=jax.ShapeDtypeStruct((n,), jnp.int32),
        compiler_params=pltpu.CompilerParams(needs_layout_passes=False),
        scratch_types=[
            pltpu.VMEM((per_w,), jnp.float32),
            pltpu.VMEM((per_w,), jnp.int32),
        ],
    )
    def k(m_hbm, out_hbm, m_v, o_v):
        wid = lax.axis_index("s") * nc + lax.axis_index("c")
        base = wid * per_w
        pltpu.sync_copy(m_hbm.at[pl.ds(base, per_w)], m_v)
        for i in range(per_w // _LANES):
            sl = pl.ds(i * _LANES, _LANES)
            o_v[sl] = (m_v[sl] != 0.0).astype(jnp.int32)
        pltpu.sync_copy(o_v, out_hbm.at[pl.ds(base, per_w)])

    return k(mask_flat)


def _copy_u_tc(U):
    Uf = U.reshape(-1, U.shape[-2], U.shape[-1])
    rows = Uf.shape[0]
    cr = 2
    n_chunks = rows // cr

    def body(u_in, u_out, buf, in_sems, out_sems):
        def in_cp(c):
            k = c % _NBUF
            return pltpu.make_async_copy(
                u_in.at[pl.ds(c * cr, cr)], buf.at[k], in_sems.at[k]
            )

        def out_cp(c):
            k = c % _NBUF
            return pltpu.make_async_copy(
                buf.at[k], u_out.at[pl.ds(c * cr, cr)], out_sems.at[k]
            )

        for c in range(n_chunks + _LAG):
            if c < n_chunks:
                if c >= _NBUF:
                    out_cp(c - _NBUF).wait()
                in_cp(c).start()
            d = c - _LAG
            if d >= 0:
                in_cp(d).wait()
                out_cp(d).start()
        for c in range(n_chunks - _NBUF, n_chunks):
            out_cp(c).wait()

    out = pl.pallas_call(
        body,
        in_specs=[pl.BlockSpec(memory_space=pl.ANY)],
        out_specs=pl.BlockSpec(memory_space=pl.ANY),
        out_shape=jax.ShapeDtypeStruct(Uf.shape, Uf.dtype),
        scratch_shapes=[
            pltpu.VMEM((_NBUF, cr, Uf.shape[-2], Uf.shape[-1]), jnp.float32),
            pltpu.SemaphoreType.DMA((_NBUF,)),
            pltpu.SemaphoreType.DMA((_NBUF,)),
        ],
        compiler_params=pltpu.CompilerParams(
            vmem_limit_bytes=100 * 1024 * 1024,
        ),
    )(Uf)
    return out.reshape(U.shape)


def kernel(x, v, mask, U):
    mi = _booleanize_sc(mask.reshape(-1))
    oU = _copy_u_tc(U)
    mb = mi.astype(jnp.bool_).reshape(mask.shape)
    return (x, v, mb, oU)
